# Initial kernel scaffold; baseline (speedup 1.0000x reference)
import jax
import jax.numpy as jnp
from jax.experimental import pallas as pl


def kernel(x_ingredient, x_direction, cond, proj_ing_W, proj_ing_b, proj_dir_W, proj_dir_b, a_src_co, a_dst_co, a_src_ui, a_dst_ui, a_src_ct, a_dst_ct, a_src_pw, a_dst_pw, a_src_fl, a_dst_fl, k_lin_W, k_lin_b, q, fc_mu_W, fc_mu_b, fc_lv_W, fc_lv_b, ei_co, ei_ui, ei_ct, ei_pw, ei_fl):
    raise NotImplementedError("write your pallas kernel here")


# speedup vs baseline: 16.0830x; 16.0830x over previous
import functools
import jax
import jax.numpy as jnp
from jax.experimental import pallas as pl
from jax.experimental.pallas import tpu as pltpu

H = 8
D = 16
HID = 128

_BLK = 5000


def _proj_body(x_ref, w_ref, a_ref, h_ref, al_ref):
    h = jnp.dot(x_ref[...], w_ref[...], preferred_element_type=jnp.float32)
    h_ref[...] = h
    al_ref[...] = jnp.dot(h, a_ref[...], preferred_element_type=jnp.float32)


def _project(x, W, Amat):
    n = x.shape[0]
    k = Amat.shape[1]
    grid = (n // _BLK,)
    return pl.pallas_call(
        _proj_body,
        grid=grid,
        in_specs=[
            pl.BlockSpec((_BLK, 128), lambda i: (i, 0)),
            pl.BlockSpec((128, 128), lambda i: (0, 0)),
            pl.BlockSpec((128, k), lambda i: (0, 0)),
        ],
        out_specs=[
            pl.BlockSpec((_BLK, 128), lambda i: (i, 0)),
            pl.BlockSpec((_BLK, k), lambda i: (i, 0)),
        ],
        out_shape=[
            jax.ShapeDtypeStruct((n, 128), jnp.float32),
            jax.ShapeDtypeStruct((n, k), jnp.float32),
        ],
    )(x, W.T, Amat)


def _amat(*avs):
    cols = []
    for a in avs:
        m = jnp.zeros((HID, H), jnp.float32)
        idx = jnp.arange(HID)
        m = m.at[idx, idx // D].set(a.reshape(HID))
        cols.append(m)
    return jnp.concatenate(cols, axis=1)


def _edge_phase(h_src, alpha_src, alpha_dst, src, dst, num_dst):
    al = alpha_src[src] + alpha_dst[dst]
    e = jnp.exp(jnp.where(al > 0, al, 0.2 * al))
    den = jax.ops.segment_sum(e, dst, num_segments=num_dst)
    msg = h_src[src].reshape(-1, H, D) * e[:, :, None]
    num = jax.ops.segment_sum(msg.reshape(-1, HID), dst, num_segments=num_dst)
    return jax.nn.relu(num / (den.repeat(D, axis=1) + 1e-16))


def kernel(x_ingredient, x_direction, cond, proj_ing_W, proj_ing_b, proj_dir_W,
           proj_dir_b, a_src_co, a_dst_co, a_src_ui, a_dst_ui, a_src_ct,
           a_dst_ct, a_src_pw, a_dst_pw, a_src_fl, a_dst_fl, k_lin_W, k_lin_b,
           q, fc_mu_W, fc_mu_b, fc_lv_W, fc_lv_b, ei_co, ei_ui, ei_ct, ei_pw,
           ei_fl):
    n_ing = x_ingredient.shape[0]
    n_dir = x_direction.shape[0]

    amat_ing = _amat(a_src_co, a_src_ui, a_dst_ct)
    amat_dir = _amat(a_dst_co, a_dst_ui, a_src_ct, a_src_pw, a_dst_pw,
                     a_src_fl, a_dst_fl)
    h_ing, al_ing = _project(x_ingredient, proj_ing_W, amat_ing)
    h_dir, al_dir = _project(x_direction, proj_dir_W, amat_dir)
    asrc_co, asrc_ui, adst_ct = (al_ing[:, 0:8], al_ing[:, 8:16],
                                 al_ing[:, 16:24])
    (adst_co, adst_ui, asrc_ct, asrc_pw, adst_pw, asrc_fl,
     adst_fl) = [al_dir[:, 8 * i:8 * i + 8] for i in range(7)]

    o_co = _edge_phase(h_ing, asrc_co, adst_co, ei_co[0], ei_co[1], n_dir)
    o_ui = _edge_phase(h_ing, asrc_ui, adst_ui, ei_ui[0], ei_ui[1], n_dir)
    o_ct = _edge_phase(h_dir, asrc_ct, adst_ct, ei_ct[0], ei_ct[1], n_ing)
    o_pw = _edge_phase(h_dir, asrc_pw, adst_pw, ei_pw[0], ei_pw[1], n_dir)
    o_fl = _edge_phase(h_dir, asrc_fl, adst_fl, ei_fl[0], ei_fl[1], n_dir)

    outs = [o_co, o_ui, o_pw, o_fl]
    ks = jnp.stack([jnp.tanh(o @ k_lin_W.T + k_lin_b).mean(0) for o in outs])
    score = (ks * q).sum(-1)
    attn = jax.nn.softmax(score)
    pool_dir = sum(a * o.mean(0) for a, o in zip(attn, outs))
    pool_ing = o_ct.mean(0)

    g = jnp.concatenate([pool_ing, pool_dir, cond])[None, :]
    mu = g @ fc_mu_W.T + fc_mu_b
    logvar = g @ fc_lv_W.T + fc_lv_b
    return (mu, logvar)
